# Initial kernel scaffold; baseline (speedup 1.0000x reference)
import jax
import jax.numpy as jnp
from jax.experimental import pallas as pl


def kernel(pts, vertices, vertices_0, faces, v_faces):
    raise NotImplementedError("write your pallas kernel here")


# speedup vs baseline: 3.4521x; 3.4521x over previous
import functools

import jax
import jax.numpy as jnp
from jax import lax
from jax.experimental import pallas as pl
from jax.experimental.pallas import tpu as pltpu
from jax.experimental.pallas import tpu_sc as plsc

_NC, _NS, _L = 2, 16, 16
_NW = _NC * _NS


def _nearest_body(pts_ref, verts_ref, vi_ref, look_ref, dir_ref):
    px = pts_ref[:, 0:1]
    py = pts_ref[:, 1:2]
    pz = pts_ref[:, 2:3]
    vx = verts_ref[0:1, :]
    vy = verts_ref[1:2, :]
    vz = verts_ref[2:3, :]
    dx = px - vx
    dy = py - vy
    dz = pz - vz
    d2 = dx * dx + dy * dy + dz * dz
    m = jnp.min(d2, axis=1, keepdims=True)
    iota = lax.broadcasted_iota(jnp.int32, d2.shape, 1)
    big = jnp.int32(2 ** 30)
    vi = jnp.min(jnp.where(d2 == m, iota, big), axis=1, keepdims=True)
    oh = iota == vi
    zero = jnp.float32(0.0)
    nx = jnp.sum(jnp.where(oh, vx, zero), axis=1, keepdims=True)
    ny = jnp.sum(jnp.where(oh, vy, zero), axis=1, keepdims=True)
    nz = jnp.sum(jnp.where(oh, vz, zero), axis=1, keepdims=True)
    lx = px - nx
    ly = py - ny
    lz = pz - nz
    look = jnp.concatenate([lx, ly, lz], axis=1)
    dist = jnp.sqrt(lx * lx + ly * ly + lz * lz)
    dirv = look / jnp.maximum(dist, jnp.float32(1e-8))
    vi_ref[...] = vi
    look_ref[...] = look
    dir_ref[...] = dirv


def _tc_nearest(pts, verts_t):
    n = pts.shape[0]
    vp = verts_t.shape[1]
    p = 256
    return pl.pallas_call(
        _nearest_body,
        grid=(n // p,),
        in_specs=[
            pl.BlockSpec((p, 3), lambda i: (i, 0)),
            pl.BlockSpec((3, vp), lambda i: (0, 0)),
        ],
        out_specs=[
            pl.BlockSpec((p, 1), lambda i: (i, 0)),
            pl.BlockSpec((p, 3), lambda i: (i, 0)),
            pl.BlockSpec((p, 3), lambda i: (i, 0)),
        ],
        out_shape=[
            jax.ShapeDtypeStruct((n, 1), jnp.int32),
            jax.ShapeDtypeStruct((n, 3), jnp.float32),
            jax.ShapeDtypeStruct((n, 3), jnp.float32),
        ],
    )(pts, verts_t)


def _rsqrt16(x):
    xh = x * jnp.float32(0.5)
    i = plsc.bitcast(x, jnp.int32)
    i = jnp.int32(0x5F3759DF) - lax.shift_right_logical(i, 1)
    y = plsc.bitcast(i, jnp.float32)
    for _ in range(4):
        y = y * (jnp.float32(1.5) - xh * y * y)
    return y


def _cross(ax, ay, az, bx, by, bz):
    return ay * bz - az * by, az * bx - ax * bz, ax * by - ay * bx


def _mesh():
    return plsc.VectorSubcoreMesh(core_axis_name="c", subcore_axis_name="s",
                                  num_cores=_NC, num_subcores=_NS)


_SC_PARAMS = pltpu.CompilerParams(needs_layout_passes=False)


def _wid():
    return lax.axis_index("s") * _NC + lax.axis_index("c")


def _sc_face_normals(faces_flat, verts_flat, fp, v):
    per = fp // _NW
    ng = per // _L

    @functools.partial(
        pl.kernel,
        out_type=jax.ShapeDtypeStruct((3 * fp,), jnp.float32),
        mesh=_mesh(),
        compiler_params=_SC_PARAMS,
        scratch_types=[
            pltpu.VMEM((3 * v,), jnp.float32),
            pltpu.VMEM((3 * per,), jnp.int32),
            pltpu.VMEM((3 * per,), jnp.float32),
        ],
    )
    def k(faces_h, verts_h, out_h, verts_v, faces_v, out_v):
        base = _wid() * per
        pltpu.sync_copy(verts_h, verts_v)
        for c in range(3):
            pltpu.sync_copy(faces_h.at[pl.ds(c * fp + base, per)],
                            faces_v.at[pl.ds(c * per, per)])
        for g in range(ng):
            ia = faces_v[pl.ds(0 * per + g * _L, _L)]
            ib = faces_v[pl.ds(1 * per + g * _L, _L)]
            ic = faces_v[pl.ds(2 * per + g * _L, _L)]
            ax = plsc.load_gather(verts_v, [ia])
            ay = plsc.load_gather(verts_v, [ia + v])
            az = plsc.load_gather(verts_v, [ia + 2 * v])
            bx = plsc.load_gather(verts_v, [ib])
            by = plsc.load_gather(verts_v, [ib + v])
            bz = plsc.load_gather(verts_v, [ib + 2 * v])
            cx = plsc.load_gather(verts_v, [ic])
            cy = plsc.load_gather(verts_v, [ic + v])
            cz = plsc.load_gather(verts_v, [ic + 2 * v])
            nx, ny, nz = _cross(bx - ax, by - ay, bz - az,
                                cx - ax, cy - ay, cz - az)
            inv = _rsqrt16(nx * nx + ny * ny + nz * nz)
            out_v[pl.ds(0 * per + g * _L, _L)] = nx * inv
            out_v[pl.ds(1 * per + g * _L, _L)] = ny * inv
            out_v[pl.ds(2 * per + g * _L, _L)] = nz * inv
        for c in range(3):
            pltpu.sync_copy(out_v.at[pl.ds(c * per, per)],
                            out_h.at[pl.ds(c * fp + base, per)])

    return k(faces_flat, verts_flat)


def _sc_pick_face(vf_flat, fn_flat, vi, dir_flat, n, v, fp, j_width):
    per = n // _NW
    ng = per // _L

    @functools.partial(
        pl.kernel,
        out_type=(
            jax.ShapeDtypeStruct((n,), jnp.int32),
            jax.ShapeDtypeStruct((3 * n,), jnp.float32),
        ),
        mesh=_mesh(),
        compiler_params=_SC_PARAMS,
        scratch_types=[
            pltpu.VMEM((j_width * v,), jnp.int32),
            pltpu.VMEM((3 * fp,), jnp.float32),
            pltpu.VMEM((per,), jnp.int32),
            pltpu.VMEM((3 * per,), jnp.float32),
            pltpu.VMEM((per,), jnp.int32),
            pltpu.VMEM((3 * per,), jnp.float32),
        ],
    )
    def k(vf_h, fn_h, vi_h, dir_h, f_out_h, nrm_out_h,
          vf_v, fn_v, vi_v, dir_v, f_v, nrm_v):
        base = _wid() * per
        pltpu.sync_copy(vf_h, vf_v)
        pltpu.sync_copy(fn_h, fn_v)
        pltpu.sync_copy(vi_h.at[pl.ds(base, per)], vi_v)
        for c in range(3):
            pltpu.sync_copy(dir_h.at[pl.ds(c * n + base, per)],
                            dir_v.at[pl.ds(c * per, per)])
        for g in range(ng):
            sl = pl.ds(g * _L, _L)
            vi_g = vi_v[sl]
            dx = dir_v[pl.ds(0 * per + g * _L, _L)]
            dy = dir_v[pl.ds(1 * per + g * _L, _L)]
            dz = dir_v[pl.ds(2 * per + g * _L, _L)]
            best = jnp.full((_L,), 3e38, jnp.float32)
            bf = jnp.zeros((_L,), jnp.int32)
            bnx = jnp.zeros((_L,), jnp.float32)
            bny = jnp.zeros((_L,), jnp.float32)
            bnz = jnp.zeros((_L,), jnp.float32)
            for j in range(j_width):
                fid = plsc.load_gather(vf_v, [vi_g + j * v])
                nx = plsc.load_gather(fn_v, [fid])
                ny = plsc.load_gather(fn_v, [fid + fp])
                nz = plsc.load_gather(fn_v, [fid + 2 * fp])
                cosv = dx * nx + dy * ny + dz * nz
                ad = jnp.float32(1.0) - jnp.abs(cosv)
                upd = ad < best
                best = jnp.where(upd, ad, best)
                bf = jnp.where(upd, fid, bf)
                bnx = jnp.where(upd, nx, bnx)
                bny = jnp.where(upd, ny, bny)
                bnz = jnp.where(upd, nz, bnz)
            f_v[sl] = bf
            nrm_v[pl.ds(0 * per + g * _L, _L)] = bnx
            nrm_v[pl.ds(1 * per + g * _L, _L)] = bny
            nrm_v[pl.ds(2 * per + g * _L, _L)] = bnz
        pltpu.sync_copy(f_v, f_out_h.at[pl.ds(base, per)])
        for c in range(3):
            pltpu.sync_copy(nrm_v.at[pl.ds(c * per, per)],
                            nrm_out_h.at[pl.ds(c * n + base, per)])

    return k(vf_flat, fn_flat, vi, dir_flat)


def _sc_project(faces_flat, vertsi_flat, verts0_flat, f_star, pts_flat,
                look_flat, nrm_flat, n, v, fp):
    per = n // _NW
    ng = per // _L

    @functools.partial(
        pl.kernel,
        out_type=jax.ShapeDtypeStruct((3 * n,), jnp.float32),
        mesh=_mesh(),
        compiler_params=_SC_PARAMS,
        scratch_types=[
            pltpu.VMEM((3 * fp,), jnp.int32),
            pltpu.VMEM((3 * v,), jnp.float32),
            pltpu.VMEM((3 * v,), jnp.float32),
            pltpu.VMEM((per,), jnp.int32),
            pltpu.VMEM((3 * per,), jnp.float32),
            pltpu.VMEM((3 * per,), jnp.float32),
            pltpu.VMEM((3 * per,), jnp.float32),
            pltpu.VMEM((3 * per,), jnp.float32),
        ],
    )
    def k(faces_h, vertsi_h, verts0_h, f_h, pts_h, look_h, nrm_h, out_h,
          faces_v, vertsi_v, verts0_v, f_v, pts_v, look_v, nrm_v, out_v):
        base = _wid() * per
        pltpu.sync_copy(faces_h, faces_v)
        pltpu.sync_copy(vertsi_h, vertsi_v)
        pltpu.sync_copy(verts0_h, verts0_v)
        pltpu.sync_copy(f_h.at[pl.ds(base, per)], f_v)
        for c in range(3):
            pltpu.sync_copy(pts_h.at[pl.ds(c * n + base, per)],
                            pts_v.at[pl.ds(c * per, per)])
            pltpu.sync_copy(look_h.at[pl.ds(c * n + base, per)],
                            look_v.at[pl.ds(c * per, per)])
            pltpu.sync_copy(nrm_h.at[pl.ds(c * n + base, per)],
                            nrm_v.at[pl.ds(c * per, per)])
        one = jnp.float32(1.0)
        for g in range(ng):
            sl = pl.ds(g * _L, _L)
            fid = f_v[sl]
            ia = plsc.load_gather(faces_v, [fid])
            ib = plsc.load_gather(faces_v, [fid + fp])
            ic = plsc.load_gather(faces_v, [fid + 2 * fp])
            px = pts_v[pl.ds(0 * per + g * _L, _L)]
            py = pts_v[pl.ds(1 * per + g * _L, _L)]
            pz = pts_v[pl.ds(2 * per + g * _L, _L)]
            lx = look_v[pl.ds(0 * per + g * _L, _L)]
            ly = look_v[pl.ds(1 * per + g * _L, _L)]
            lz = look_v[pl.ds(2 * per + g * _L, _L)]
            nx = nrm_v[pl.ds(0 * per + g * _L, _L)]
            ny = nrm_v[pl.ds(1 * per + g * _L, _L)]
            nz = nrm_v[pl.ds(2 * per + g * _L, _L)]
            pd = nx * lx + ny * ly + nz * lz
            qx = px - nx * pd
            qy = py - ny * pd
            qz = pz - nz * pd
            tri = []
            for idv in (ia, ib, ic):
                tx = plsc.load_gather(vertsi_v, [idv])
                ty = plsc.load_gather(vertsi_v, [idv + v])
                tz = plsc.load_gather(vertsi_v, [idv + 2 * v])
                tri.append((tx, ty, tz))
            us = []
            dist_q = []
            for (tx, ty, tz) in tri:
                ddx = tx - qx
                ddy = ty - qy
                ddz = tz - qz
                d2 = ddx * ddx + ddy * ddy + ddz * ddz
                dn = jnp.maximum(d2 * _rsqrt16(d2), jnp.float32(1e-8))
                r = one / dn
                us.append((ddx * r, ddy * r, ddz * r))
                dist_q.append(dn)
            ws = []
            for kk in range(3):
                ur = us[(kk + 1) % 3]
                ul = us[(kk + 2) % 3]
                cpx, cpy, cpz = _cross(ur[0], ur[1], ur[2],
                                       ul[0], ul[1], ul[2])
                sg = jnp.sign(cpx * qx + cpy * qy + cpz * qz)
                s2 = cpx * cpx + cpy * cpy + cpz * cpz
                sin_t = s2 * _rsqrt16(s2)
                ws.append(sin_t * dist_q[(kk + 2) % 3] * dist_q[(kk + 1) % 3] * sg)
            wsum = ws[0] + ws[1] + ws[2]
            w0 = ws[0] / wsum
            w1 = ws[1] / wsum
            w2 = ws[2] / wsum
            tri0 = []
            for idv in (ia, ib, ic):
                tx = plsc.load_gather(verts0_v, [idv])
                ty = plsc.load_gather(verts0_v, [idv + v])
                tz = plsc.load_gather(verts0_v, [idv + 2 * v])
                tri0.append((tx, ty, tz))
            n0x, n0y, n0z = _cross(
                tri0[1][0] - tri0[0][0], tri0[1][1] - tri0[0][1],
                tri0[1][2] - tri0[0][2],
                tri0[2][0] - tri0[0][0], tri0[2][1] - tri0[0][1],
                tri0[2][2] - tri0[0][2])
            inv0 = _rsqrt16(n0x * n0x + n0y * n0y + n0z * n0z)
            n0x = n0x * inv0
            n0y = n0y * inv0
            n0z = n0z * inv0
            ox = w0 * tri0[0][0] + w1 * tri0[1][0] + w2 * tri0[2][0] + n0x * pd
            oy = w0 * tri0[0][1] + w1 * tri0[1][1] + w2 * tri0[2][1] + n0y * pd
            oz = w0 * tri0[0][2] + w1 * tri0[1][2] + w2 * tri0[2][2] + n0z * pd
            out_v[pl.ds(0 * per + g * _L, _L)] = ox
            out_v[pl.ds(1 * per + g * _L, _L)] = oy
            out_v[pl.ds(2 * per + g * _L, _L)] = oz
        for c in range(3):
            pltpu.sync_copy(out_v.at[pl.ds(c * per, per)],
                            out_h.at[pl.ds(c * n + base, per)])

    return k(faces_flat, vertsi_flat, verts0_flat, f_star, pts_flat,
             look_flat, nrm_flat)


def kernel(pts, vertices, vertices_0, faces, v_faces):
    n = pts.shape[0]
    v = vertices.shape[1]
    f = faces.shape[0]
    j_width = v_faces.shape[1]
    verts_i = vertices.reshape(v, 3)

    vp = ((v + 127) // 128) * 128
    verts_t = jnp.concatenate(
        [verts_i.T, jnp.full((3, vp - v), 1e6, jnp.float32)], axis=1)
    vi2, look, dirv = _tc_nearest(pts, verts_t)
    vi = vi2.reshape(n)

    fp = ((f + _NW * _L - 1) // (_NW * _L)) * (_NW * _L)
    faces_pad = jnp.concatenate(
        [faces.astype(jnp.int32), jnp.zeros((fp - f, 3), jnp.int32)], axis=0)
    faces_flat = faces_pad.T.reshape(-1)
    vertsi_flat = verts_i.T.reshape(-1)
    verts0_flat = vertices_0.T.reshape(-1)
    vf_flat = v_faces.astype(jnp.int32).T.reshape(-1)
    pts_flat = pts.T.reshape(-1)
    look_flat = look.T.reshape(-1)
    dir_flat = dirv.T.reshape(-1)

    fn_flat = _sc_face_normals(faces_flat, vertsi_flat, fp, v)
    f_star, nrm_flat = _sc_pick_face(vf_flat, fn_flat, vi, dir_flat,
                                     n, v, fp, j_width)
    out_flat = _sc_project(faces_flat, vertsi_flat, verts0_flat, f_star,
                           pts_flat, look_flat, nrm_flat, n, v, fp)
    return out_flat.reshape(3, n).T
